# baseline (speedup 1.0000x reference)
import functools

import jax
import jax.numpy as jnp
from jax import lax
from jax.experimental import pallas as pl
from jax.experimental.pallas import tpu as pltpu
from jax.experimental.pallas import tpu_sc as plsc

N_NODES = 100000
DIM = 128
NROWS = 65
NC, NS = 2, 16
NW = NC * NS
CHUNK = 400
NCHUNKS = N_NODES // CHUNK
MAXK = (NCHUNKS + NW - 1) // NW
HALF = CHUNK // 2


def _make_sc_gather():
    mesh = plsc.VectorSubcoreMesh(core_axis_name="c", subcore_axis_name="s")

    @functools.partial(
        pl.kernel,
        out_type=jax.ShapeDtypeStruct((N_NODES * DIM,), jnp.float32),
        mesh=mesh,
        compiler_params=pltpu.CompilerParams(needs_layout_passes=False),
        scratch_types=[
            pltpu.VMEM((NROWS * DIM,), jnp.float32),
            pltpu.VMEM((CHUNK,), jnp.int32),
            pltpu.VMEM((CHUNK * DIM,), jnp.float32),
            pltpu.VMEM((CHUNK * DIM,), jnp.float32),
            pltpu.SemaphoreType.DMA,
            pltpu.SemaphoreType.DMA,
        ],
    )
    def sc_gather(deg_hbm, table_hbm, out_hbm,
                  table_v, idx_v, rows0, rows1, sem0, sem1):
        wid = lax.axis_index("s") * NC + lax.axis_index("c")
        nk = (NCHUNKS - wid + NW - 1) // NW
        rows, sems = (rows0, rows1), (sem0, sem1)

        pltpu.sync_copy(table_hbm, table_v)

        lane = lax.iota(jnp.int32, 16)

        def gather_rows(b):
            @plsc.parallel_loop(0, CHUNK // 16, unroll=1)
            def _(g):
                d_slice = idx_v[pl.ds(g * 16, 16)]
                base_r = g * (16 * DIM)
                for rr in range(16):
                    d_bcast = lax.gather(
                        d_slice,
                        jnp.full((16, 1), rr, jnp.int32),
                        lax.GatherDimensionNumbers(
                            offset_dims=(), collapsed_slice_dims=(0,),
                            start_index_map=(0,)),
                        slice_sizes=(1,),
                        mode=lax.GatherScatterMode.PROMISE_IN_BOUNDS)
                    src0 = d_bcast * DIM + lane
                    for j in range(DIM // 16):
                        v = plsc.load_gather(table_v, [src0 + j * 16])
                        rows[b][pl.ds(base_r + rr * DIM + j * 16, 16)] = v

        def process(k, b):
            base = (wid + k * NW) * CHUNK

            pl.when(k >= 2)(lambda: pltpu.make_async_copy(
                rows[b], out_hbm.at[pl.ds(0, CHUNK * DIM)], sems[b]).wait())

            pltpu.sync_copy(deg_hbm.at[pl.ds(base, CHUNK)], idx_v)

            gather_rows(b)
            pltpu.async_copy(
                rows[b], out_hbm.at[pl.ds(base * DIM, CHUNK * DIM)], sems[b])

        def outer(i, _):
            for b in range(2):
                k = i * 2 + b
                pl.when(k < nk)(lambda k=k, b=b: process(k, b))
            return 0

        lax.fori_loop(0, (MAXK + 1) // 2, outer, 0)

        for b in range(2):
            pltpu.make_async_copy(
                rows[b], out_hbm.at[pl.ds(0, CHUNK * DIM)], sems[b]).wait()

    return sc_gather


_sc_gather = _make_sc_gather()


def kernel(degrees, table):
    out = _sc_gather(degrees.astype(jnp.int32), table.reshape(-1))
    return out.reshape(N_NODES, DIM)


# speedup vs baseline: 1.3703x; 1.3703x over previous
import functools

import jax
import jax.numpy as jnp
from jax import lax
from jax.experimental import pallas as pl
from jax.experimental.pallas import tpu as pltpu
from jax.experimental.pallas import tpu_sc as plsc

N_NODES = 100000
DIM = 128
NROWS = 65
NC, NS = 2, 16
NW = NC * NS
CHUNK = 400
NCHUNKS = N_NODES // CHUNK
MAXK = (NCHUNKS + NW - 1) // NW
HALF = CHUNK // 2


def _make_sc_gather():
    mesh = plsc.VectorSubcoreMesh(core_axis_name="c", subcore_axis_name="s")

    @functools.partial(
        pl.kernel,
        out_type=jax.ShapeDtypeStruct((N_NODES * DIM,), jnp.float32),
        mesh=mesh,
        compiler_params=pltpu.CompilerParams(needs_layout_passes=False),
        scratch_types=[
            pltpu.VMEM((NROWS, DIM), jnp.float32),
            pltpu.VMEM((CHUNK,), jnp.int32),
            pltpu.VMEM((CHUNK * DIM,), jnp.float32),
            pltpu.VMEM((CHUNK * DIM,), jnp.float32),
            pltpu.SemaphoreType.DMA,
            pltpu.SemaphoreType.DMA,
        ],
    )
    def sc_gather(deg_hbm, table_hbm, out_hbm,
                  table_v, idx_v, rows0, rows1, sem0, sem1):
        wid = lax.axis_index("s") * NC + lax.axis_index("c")
        nk = (NCHUNKS - wid + NW - 1) // NW
        rows, sems = (rows0, rows1), (sem0, sem1)

        pltpu.sync_copy(table_hbm, table_v)

        lane = lax.iota(jnp.int32, 16)

        def gather_rows(b):
            @plsc.parallel_loop(0, CHUNK, unroll=4)
            def _(r):
                d_slice = idx_v[pl.ds((r >> 4) << 4, 16)]
                d_bcast = lax.gather(
                    d_slice,
                    jnp.full((16, 1), 0, jnp.int32) + (r & 15),
                    lax.GatherDimensionNumbers(
                        offset_dims=(), collapsed_slice_dims=(0,),
                        start_index_map=(0,)),
                    slice_sizes=(1,),
                    mode=lax.GatherScatterMode.PROMISE_IN_BOUNDS)
                for j in range(DIM // 16):
                    v = plsc.load_gather(table_v, [d_bcast, lane + j * 16])
                    rows[b][pl.ds(r * DIM + j * 16, 16)] = v

        def process(k, b):
            base = (wid + k * NW) * CHUNK

            pl.when(k >= 2)(lambda: pltpu.make_async_copy(
                rows[b], out_hbm.at[pl.ds(0, CHUNK * DIM)], sems[b]).wait())

            pltpu.sync_copy(deg_hbm.at[pl.ds(base, CHUNK)], idx_v)

            gather_rows(b)
            pltpu.async_copy(
                rows[b], out_hbm.at[pl.ds(base * DIM, CHUNK * DIM)], sems[b])

        def outer(i, _):
            for b in range(2):
                k = i * 2 + b
                pl.when(k < nk)(lambda k=k, b=b: process(k, b))
            return 0

        lax.fori_loop(0, (MAXK + 1) // 2, outer, 0)

        for b in range(2):
            pltpu.make_async_copy(
                rows[b], out_hbm.at[pl.ds(0, CHUNK * DIM)], sems[b]).wait()

    return sc_gather


_sc_gather = _make_sc_gather()


def kernel(degrees, table):
    out = _sc_gather(degrees.astype(jnp.int32), table)
    return out.reshape(N_NODES, DIM)
